# baseline (speedup 1.0000x reference)
import functools

import jax
import jax.numpy as jnp
from jax import lax
from jax.experimental import pallas as pl
from jax.experimental.pallas import tpu as pltpu
from jax.experimental.pallas import tpu_sc as plsc

_ROWS = 8192
_COLS = 4
_LANES = 16
_TC = 128
_PR = _ROWS * _COLS // _TC


def _sc_row_permute(index, phys):
    info = plsc.get_sparse_core_info()
    nc, ns = 1, info.num_subcores
    nw = nc * ns
    rpw = _PR // nw

    mesh = plsc.VectorSubcoreMesh(core_axis_name="c", subcore_axis_name="s",
                                  num_cores=nc)

    @functools.partial(
        pl.kernel,
        mesh=mesh,
        out_type=jax.ShapeDtypeStruct((_PR, _TC), jnp.float32),
        scratch_types=[
            pltpu.VMEM((_COLS,), jnp.int32),
            pltpu.VMEM((rpw,), jnp.int32),
            pltpu.VMEM((rpw, _TC), jnp.float32),
            pltpu.SemaphoreType.DMA,
        ],
        compiler_params=pltpu.CompilerParams(needs_layout_passes=False),
    )
    def k(idx_hbm, p_hbm, q_hbm, idx_v, sidx_v, rows_v, sem):
        wid = lax.axis_index("s") * nc + lax.axis_index("c")
        pltpu.async_copy(p_hbm.at[pl.ds(wid * rpw, rpw)], rows_v, sem)
        pltpu.sync_copy(idx_hbm, idx_v)
        lane = lax.iota(jnp.int32, _LANES)
        idx16 = plsc.load_gather(idx_v, [lane % _COLS])
        s16 = (wid * rpw + (lane // _COLS) * _COLS) + idx16
        sidx_v[...] = s16
        pltpu.make_async_copy(
            p_hbm.at[pl.ds(wid * rpw, rpw)], rows_v, sem).wait()
        pltpu.async_copy(rows_v, q_hbm.at[sidx_v], sem).wait()

    return k(index, phys)


def kernel(index, update, params):
    del params
    phys = (update.T.reshape(_COLS, _PR // _COLS, _TC)
            .transpose(1, 0, 2).reshape(_PR, _TC))
    q = _sc_row_permute(index.astype(jnp.int32), phys)
    out = (q.reshape(_PR // _COLS, _COLS, _TC)
           .transpose(1, 0, 2).reshape(_COLS, _ROWS).T)
    return out.reshape(1, _ROWS, _COLS)
